# Initial kernel scaffold; baseline (speedup 1.0000x reference)
import jax
import jax.numpy as jnp
from jax.experimental import pallas as pl


def kernel(input, partitions, W, b):
    raise NotImplementedError("write your pallas kernel here")


# speedup vs baseline: 1.2797x; 1.2797x over previous
import functools

import jax
import jax.numpy as jnp
from jax import lax
from jax.experimental import pallas as pl
from jax.experimental.pallas import tpu as pltpu
from jax.experimental.pallas import tpu_sc as plsc

N_EXPERTS = 8
TOKENS = 4096
D_IN = 1024
D_OUT = 1024

BLK = 256
NB = TOKENS // BLK
MAX_STEPS = NB + N_EXPERTS - 1

NW = 32
ROWS_PER_W = TOKENS // NW
CHUNK = 64


def _sc_mesh():
    return plsc.VectorSubcoreMesh(core_axis_name="c", subcore_axis_name="s")


def _sc_scatter_body(src_hbm, dest_hbm, out_hbm, idx_v, rows_v, sem):
    wid = lax.axis_index("s") * 2 + lax.axis_index("c")
    base = wid * ROWS_PER_W
    for c in range(ROWS_PER_W // CHUNK):
        cb = base + c * CHUNK
        pltpu.sync_copy(dest_hbm.at[pl.ds(cb, CHUNK)], idx_v)
        pltpu.sync_copy(src_hbm.at[pl.ds(cb, CHUNK)], rows_v)
        pltpu.async_copy(rows_v, out_hbm.at[idx_v], sem).wait()


def _sc_scatter_rows(src, dest):
    kern = functools.partial(
        pl.kernel,
        out_type=jax.ShapeDtypeStruct((TOKENS, D_IN), jnp.float32),
        mesh=_sc_mesh(),
        scratch_types=[
            pltpu.VMEM((CHUNK,), jnp.int32),
            pltpu.VMEM((CHUNK, D_IN), jnp.float32),
            pltpu.SemaphoreType.DMA,
        ],
    )(_sc_scatter_body)
    return kern(src, dest)


def _sc_gather_body(src_hbm, dest_hbm, out_hbm, idx_v, rows_v, sem):
    wid = lax.axis_index("s") * 2 + lax.axis_index("c")
    base = wid * ROWS_PER_W
    for c in range(ROWS_PER_W // CHUNK):
        cb = base + c * CHUNK
        pltpu.sync_copy(dest_hbm.at[pl.ds(cb, CHUNK)], idx_v)
        pltpu.async_copy(src_hbm.at[idx_v], rows_v, sem).wait()
        pltpu.sync_copy(rows_v, out_hbm.at[pl.ds(cb, CHUNK)])


def _sc_gather_rows(src, dest):
    kern = functools.partial(
        pl.kernel,
        out_type=jax.ShapeDtypeStruct((TOKENS, D_OUT), jnp.float32),
        mesh=_sc_mesh(),
        scratch_types=[
            pltpu.VMEM((CHUNK,), jnp.int32),
            pltpu.VMEM((CHUNK, D_OUT), jnp.float32),
            pltpu.SemaphoreType.DMA,
        ],
    )(_sc_gather_body)
    return kern(src, dest)


def _mm_body(meta_ref, off_ref, xs_ref, w_ref, b_ref, out_ref):
    s = pl.program_id(0)
    i = meta_ref[0, s]
    e = meta_ref[1, s]
    valid = meta_ref[2, s] == 1
    first = meta_ref[3, s] == 1

    @pl.when(first)
    def _init():
        out_ref[...] = jnp.zeros_like(out_ref)

    @pl.when(valid)
    def _compute():
        x = xs_ref[...]
        w = w_ref[0]
        y = lax.dot_general(
            x, w, (((1,), (1,)), ((), ())),
            preferred_element_type=jnp.float32,
            precision=lax.Precision.HIGHEST,
        )
        lo = off_ref[e]
        hi = off_ref[e + 1]
        j = i * BLK + lax.broadcasted_iota(jnp.int32, (BLK, 1), 0)
        mask = (j >= lo) & (j < hi)
        out_ref[...] += jnp.where(mask, y + b_ref[0, 0, :][None, :], 0.0)


def _grouped_matmul(x_sorted, W, b, meta, offsets):
    grid_spec = pltpu.PrefetchScalarGridSpec(
        num_scalar_prefetch=2,
        grid=(MAX_STEPS,),
        in_specs=[
            pl.BlockSpec((BLK, D_IN), lambda s, m, o: (m[0, s], 0)),
            pl.BlockSpec((1, D_OUT, D_IN), lambda s, m, o: (m[1, s], 0, 0)),
            pl.BlockSpec((1, 1, D_OUT), lambda s, m, o: (m[1, s], 0, 0)),
        ],
        out_specs=pl.BlockSpec((BLK, D_OUT), lambda s, m, o: (m[0, s], 0)),
    )
    return pl.pallas_call(
        _mm_body,
        grid_spec=grid_spec,
        out_shape=jax.ShapeDtypeStruct((TOKENS, D_OUT), jnp.float32),
    )(meta, offsets, x_sorted, W, b.reshape(N_EXPERTS, 1, D_OUT))


def _routing_metadata(p):
    onehot = (p[:, None] == jnp.arange(N_EXPERTS, dtype=jnp.int32)[None, :])
    counts = jnp.sum(onehot.astype(jnp.int32), axis=0)
    offsets = jnp.concatenate(
        [jnp.zeros((1,), jnp.int32), jnp.cumsum(counts).astype(jnp.int32)])
    rank = jnp.sum((jnp.cumsum(onehot.astype(jnp.int32), axis=0) - 1) * onehot,
                   axis=1)
    dest = offsets[p] + rank

    blk_start = jnp.arange(NB, dtype=jnp.int32) * BLK
    lo, hi = offsets[:-1], offsets[1:]
    present = ((lo[None, :] < (blk_start + BLK)[:, None])
               & (hi[None, :] > blk_start[:, None])
               & (counts[None, :] > 0))
    flags = present.reshape(-1)
    order = jnp.argsort(~flags, stable=True).astype(jnp.int32)
    nv = jnp.sum(flags.astype(jnp.int32))
    pos = jnp.minimum(jnp.arange(MAX_STEPS, dtype=jnp.int32), nv - 1)
    pair = order[pos]
    block_ids = pair // N_EXPERTS
    expert_ids = pair % N_EXPERTS
    valid = (jnp.arange(MAX_STEPS, dtype=jnp.int32) < nv).astype(jnp.int32)
    first = valid * jnp.concatenate(
        [jnp.ones((1,), jnp.int32),
         (block_ids[1:] != block_ids[:-1]).astype(jnp.int32)])
    meta = jnp.stack([block_ids, expert_ids, valid, first]).astype(jnp.int32)
    return dest.astype(jnp.int32), meta, offsets


def kernel(input, partitions, W, b):
    input_shape = input.shape
    x = input.reshape(-1, input_shape[-1])
    p = partitions.reshape(-1).astype(jnp.int32)

    dest, meta, offsets = _routing_metadata(p)
    x_sorted = _sc_scatter_rows(x, dest)
    out_sorted = _grouped_matmul(x_sorted, W, b, meta, offsets)
    out = _sc_gather_rows(out_sorted, dest)
    return out.reshape(tuple(input_shape[:-1]) + (W.shape[1],))


# speedup vs baseline: 1.9997x; 1.5626x over previous
import functools

import jax
import jax.numpy as jnp
from jax import lax
from jax.experimental import pallas as pl
from jax.experimental.pallas import tpu as pltpu
from jax.experimental.pallas import tpu_sc as plsc

N_EXPERTS = 8
TOKENS = 4096
D_IN = 1024
D_OUT = 1024

BLK = 256
NB = TOKENS // BLK
MAX_STEPS = NB + N_EXPERTS - 1

NW = 32
ROWS_PER_W = TOKENS // NW
CHUNK = 64


def _sc_mesh():
    return plsc.VectorSubcoreMesh(core_axis_name="c", subcore_axis_name="s")


def _sc_scatter_body(src_hbm, dest_hbm, out_hbm, idx_v, rows_v, sem):
    wid = lax.axis_index("s") * 2 + lax.axis_index("c")
    base = wid * ROWS_PER_W
    for c in range(ROWS_PER_W // CHUNK):
        cb = base + c * CHUNK
        pltpu.sync_copy(dest_hbm.at[pl.ds(cb, CHUNK)], idx_v)
        pltpu.sync_copy(src_hbm.at[pl.ds(cb, CHUNK)], rows_v)
        pltpu.async_copy(rows_v, out_hbm.at[idx_v], sem).wait()


def _sc_scatter_rows(src, dest):
    kern = functools.partial(
        pl.kernel,
        out_type=jax.ShapeDtypeStruct((TOKENS, D_IN), jnp.float32),
        mesh=_sc_mesh(),
        scratch_types=[
            pltpu.VMEM((CHUNK,), jnp.int32),
            pltpu.VMEM((CHUNK, D_IN), jnp.float32),
            pltpu.SemaphoreType.DMA,
        ],
    )(_sc_scatter_body)
    return kern(src, dest)


def _sc_gather_body(src_hbm, dest_hbm, out_hbm, idx_v, rows_v, sem):
    wid = lax.axis_index("s") * 2 + lax.axis_index("c")
    base = wid * ROWS_PER_W
    for c in range(ROWS_PER_W // CHUNK):
        cb = base + c * CHUNK
        pltpu.sync_copy(dest_hbm.at[pl.ds(cb, CHUNK)], idx_v)
        pltpu.async_copy(src_hbm.at[idx_v], rows_v, sem).wait()
        pltpu.sync_copy(rows_v, out_hbm.at[pl.ds(cb, CHUNK)])


def _sc_gather_rows(src, dest):
    kern = functools.partial(
        pl.kernel,
        out_type=jax.ShapeDtypeStruct((TOKENS, D_OUT), jnp.float32),
        mesh=_sc_mesh(),
        scratch_types=[
            pltpu.VMEM((CHUNK,), jnp.int32),
            pltpu.VMEM((CHUNK, D_OUT), jnp.float32),
            pltpu.SemaphoreType.DMA,
        ],
    )(_sc_gather_body)
    return kern(src, dest)


def _mm_body(meta_ref, off_ref, xs_ref, w_ref, b_ref, out_ref):
    s = pl.program_id(0)
    i = meta_ref[0, s]
    e = meta_ref[1, s]
    valid = meta_ref[2, s] == 1
    first = meta_ref[3, s] == 1

    @pl.when(first)
    def _init():
        out_ref[...] = jnp.zeros_like(out_ref)

    @pl.when(valid)
    def _compute():
        x = xs_ref[...]
        w = w_ref[0]
        y = lax.dot_general(
            x, w, (((1,), (1,)), ((), ())),
            preferred_element_type=jnp.float32,
        )
        lo = off_ref[e]
        hi = off_ref[e + 1]
        j = i * BLK + lax.broadcasted_iota(jnp.int32, (BLK, 1), 0)
        mask = (j >= lo) & (j < hi)
        out_ref[...] += jnp.where(mask, y + b_ref[0, 0, :][None, :], 0.0)


def _grouped_matmul(x_sorted, W, b, meta, offsets):
    grid_spec = pltpu.PrefetchScalarGridSpec(
        num_scalar_prefetch=2,
        grid=(MAX_STEPS,),
        in_specs=[
            pl.BlockSpec((BLK, D_IN), lambda s, m, o: (m[0, s], 0)),
            pl.BlockSpec((1, D_OUT, D_IN), lambda s, m, o: (m[1, s], 0, 0)),
            pl.BlockSpec((1, 1, D_OUT), lambda s, m, o: (m[1, s], 0, 0)),
        ],
        out_specs=pl.BlockSpec((BLK, D_OUT), lambda s, m, o: (m[0, s], 0)),
    )
    return pl.pallas_call(
        _mm_body,
        grid_spec=grid_spec,
        out_shape=jax.ShapeDtypeStruct((TOKENS, D_OUT), jnp.float32),
    )(meta, offsets, x_sorted, W, b.reshape(N_EXPERTS, 1, D_OUT))


def _routing_metadata(p):
    onehot = (p[:, None] == jnp.arange(N_EXPERTS, dtype=jnp.int32)[None, :])
    counts = jnp.sum(onehot.astype(jnp.int32), axis=0)
    offsets = jnp.concatenate(
        [jnp.zeros((1,), jnp.int32), jnp.cumsum(counts).astype(jnp.int32)])
    rank = jnp.sum((jnp.cumsum(onehot.astype(jnp.int32), axis=0) - 1) * onehot,
                   axis=1)
    dest = offsets[p] + rank

    blk_start = jnp.arange(NB, dtype=jnp.int32) * BLK
    lo, hi = offsets[:-1], offsets[1:]
    present = ((lo[None, :] < (blk_start + BLK)[:, None])
               & (hi[None, :] > blk_start[:, None])
               & (counts[None, :] > 0))
    flags = present.reshape(-1)
    order = jnp.argsort(~flags, stable=True).astype(jnp.int32)
    nv = jnp.sum(flags.astype(jnp.int32))
    pos = jnp.minimum(jnp.arange(MAX_STEPS, dtype=jnp.int32), nv - 1)
    pair = order[pos]
    block_ids = pair // N_EXPERTS
    expert_ids = pair % N_EXPERTS
    valid = (jnp.arange(MAX_STEPS, dtype=jnp.int32) < nv).astype(jnp.int32)
    first = valid * jnp.concatenate(
        [jnp.ones((1,), jnp.int32),
         (block_ids[1:] != block_ids[:-1]).astype(jnp.int32)])
    meta = jnp.stack([block_ids, expert_ids, valid, first]).astype(jnp.int32)
    return dest.astype(jnp.int32), meta, offsets


def kernel(input, partitions, W, b):
    input_shape = input.shape
    x = input.reshape(-1, input_shape[-1])
    p = partitions.reshape(-1).astype(jnp.int32)

    dest, meta, offsets = _routing_metadata(p)
    x_sorted = _sc_scatter_rows(x, dest)
    out_sorted = _grouped_matmul(x_sorted, W, b, meta, offsets)
    out = _sc_gather_rows(out_sorted, dest)
    return out.reshape(tuple(input_shape[:-1]) + (W.shape[1],))
